# Initial kernel scaffold; baseline (speedup 1.0000x reference)
import jax
import jax.numpy as jnp
from jax.experimental import pallas as pl


def kernel(user_ids, item_ids, user_feature_indices, user_feature_values, item_feature_indices, item_feature_values, U, I, UF, IF):
    raise NotImplementedError("write your pallas kernel here")


# speedup vs baseline: 3.7367x; 3.7367x over previous
import functools

import jax
import jax.numpy as jnp
from jax import lax
from jax.experimental import pallas as pl
from jax.experimental.pallas import tpu as pltpu
from jax.experimental.pallas import tpu_sc as plsc

B, F, D = 16384, 26, 32
H = D // 2
FP = 32

_info = plsc.get_sparse_core_info()
_NC, _NS = _info.num_cores, _info.num_subcores
NW = _NC * _NS
C = B // NW
S = 64
NSUB = C // S
FS = S * F
VS = S * FP
G = 128
NG = FS // G
assert FS % G == 0 and C % S == 0 and B % NW == 0 and S % 16 == 0


def _sc_forward(user_ids, item_ids, ufi, ufv, ifi, ifv, U, I, UF, IF):
  mesh = plsc.VectorSubcoreMesh(core_axis_name="c", subcore_axis_name="s")

  @functools.partial(
      pl.kernel,
      mesh=mesh,
      compiler_params=pltpu.CompilerParams(use_tc_tiling_on_sc=False),
      out_type=jax.ShapeDtypeStruct((B,), jnp.float32),
      scratch_types=[
          pltpu.VMEM((S,), jnp.int32),
          pltpu.VMEM((S,), jnp.int32),
          pltpu.VMEM((FS,), jnp.int32),
          pltpu.VMEM((VS,), jnp.float32),
          pltpu.VMEM((FS,), jnp.int32),
          pltpu.VMEM((VS,), jnp.float32),
          pltpu.VMEM((S, D), jnp.float32),
          pltpu.VMEM((S, D), jnp.float32),
          pltpu.VMEM((FS, D), jnp.float32),
          pltpu.VMEM((FS, D), jnp.float32),
          pltpu.VMEM((S,), jnp.float32),
          pltpu.SemaphoreType.DMA,
      ],
  )
  def k(uid_h, iid_h, ufi_h, ufv_h, ifi_h, ifv_h, U_h, I_h, UF_h, IF_h,
        out_h, uids_v, iids_v, ufi_v, ufv_v, ifi_v, ifv_v,
        urows_v, irows_v, ufrows_v, ifrows_v, out_v, sem):
    wid = lax.axis_index("s") * _NC + lax.axis_index("c")
    lane_iota = lax.iota(jnp.int32, 16)

    def sub(j, carry):
      base = wid * C + j * S
      fbase = base * F
      vbase = base * FP
      pltpu.sync_copy(uid_h.at[pl.ds(base, S)], uids_v)
      pltpu.sync_copy(iid_h.at[pl.ds(base, S)], iids_v)
      pltpu.sync_copy(ufi_h.at[pl.ds(fbase, FS)], ufi_v)
      pltpu.sync_copy(ufv_h.at[pl.ds(vbase, VS)], ufv_v)
      pltpu.sync_copy(ifi_h.at[pl.ds(fbase, FS)], ifi_v)
      pltpu.sync_copy(ifv_h.at[pl.ds(vbase, VS)], ifv_v)
      cps = [pltpu.async_copy(U_h.at[uids_v], urows_v, sem),
             pltpu.async_copy(I_h.at[iids_v], irows_v, sem)]
      for r in range(NG):
        sl = pl.ds(r * G, G)
        cps.append(pltpu.async_copy(UF_h.at[ufi_v.at[sl]], ufrows_v.at[sl], sem))
        cps.append(pltpu.async_copy(IF_h.at[ifi_v.at[sl]], ifrows_v.at[sl], sem))
      for cp in cps:
        cp.wait()

      def group(bg, carry2):
        def lane(l, acc):
          b = bg * 16 + l
          p0 = b * F
          v0 = b * FP
          u0 = urows_v[b, 0:H]
          u1 = urows_v[b, H:D]
          i0 = irows_v[b, 0:H]
          i1 = irows_v[b, H:D]
          uw0 = ufv_v[pl.ds(v0, 16)]
          uw1 = ufv_v[pl.ds(v0 + 16, 16)]
          iw0 = ifv_v[pl.ds(v0, 16)]
          iw1 = ifv_v[pl.ds(v0 + 16, 16)]
          for f in range(F):
            p = p0 + f
            wu = uw0[f] if f < 16 else uw1[f - 16]
            u0 = u0 + ufrows_v[p, 0:H] * wu
            u1 = u1 + ufrows_v[p, H:D] * wu
            wi = iw0[f] if f < 16 else iw1[f - 16]
            i0 = i0 + ifrows_v[p, 0:H] * wi
            i1 = i1 + ifrows_v[p, H:D] * wi
          prod = u0 * i0 + u1 * i1
          for sh in (8, 4, 2, 1):
            prod = prod + prod[lane_iota ^ sh]
          return jnp.where(lane_iota == l, prod, acc)

        acc = lax.fori_loop(0, 16, lane, jnp.zeros((16,), jnp.float32))
        out_v[pl.ds(bg * 16, 16)] = acc
        return carry2

      lax.fori_loop(0, S // 16, group, 0)
      pltpu.sync_copy(out_v, out_h.at[pl.ds(base, S)])
      return carry

    lax.fori_loop(0, NSUB, sub, 0)

  return k(user_ids, item_ids, ufi, ufv, ifi, ifv, U, I, UF, IF)


def kernel(user_ids, item_ids, user_feature_indices, user_feature_values,
           item_feature_indices, item_feature_values, U, I, UF, IF):
  pad = ((0, 0), (0, FP - F))
  return _sc_forward(
      user_ids.astype(jnp.int32),
      item_ids.astype(jnp.int32),
      user_feature_indices.astype(jnp.int32).reshape(-1),
      jnp.pad(user_feature_values, pad).reshape(-1),
      item_feature_indices.astype(jnp.int32).reshape(-1),
      jnp.pad(item_feature_values, pad).reshape(-1),
      U, I, UF, IF)
